# Initial kernel scaffold; baseline (speedup 1.0000x reference)
import jax
import jax.numpy as jnp
from jax.experimental import pallas as pl


def kernel(token_ids, weight):
    raise NotImplementedError("write your pallas kernel here")


# speedup vs baseline: 1.5165x; 1.5165x over previous
import functools

import jax
import jax.numpy as jnp
from jax import lax
from jax.experimental import pallas as pl
from jax.experimental.pallas import tpu as pltpu
from jax.experimental.pallas import tpu_sc as plsc

_NUM_CORES = 2
_NUM_SUBCORES = 16
_NUM_WORKERS = _NUM_CORES * _NUM_SUBCORES

_CH = 128
_GRP = 2048


@functools.lru_cache(maxsize=None)
def _make_lookup(num_emb, dim, batch):
    b_per_w = batch // _NUM_WORKERS
    n_grp = b_per_w // _GRP
    dmas_per_grp = _GRP // _CH
    mesh = plsc.VectorSubcoreMesh(core_axis_name="c", subcore_axis_name="s")

    @functools.partial(
        pl.kernel,
        out_type=jax.ShapeDtypeStruct((batch, dim), jnp.float32),
        mesh=mesh,
        scratch_types=[
            pltpu.VMEM((b_per_w,), jnp.int32),
            pltpu.VMEM((_GRP, dim), jnp.float32),
            pltpu.SemaphoreType.DMA,
        ],
        compiler_params=pltpu.CompilerParams(use_tc_tiling_on_sc=False),
    )
    def lookup(ids_hbm, table_hbm, out_hbm, idx_v, rows_v, sem):
        wid = lax.axis_index("s") * _NUM_CORES + lax.axis_index("c")
        base = wid * b_per_w
        pltpu.sync_copy(ids_hbm.at[pl.ds(base, b_per_w)], idx_v)

        def grp_body(g, carry):
            copies = []
            for j in range(dmas_per_grp):
                copies.append(pltpu.async_copy(
                    table_hbm.at[idx_v.at[pl.ds(g * _GRP + j * _CH, _CH)]],
                    rows_v.at[pl.ds(j * _CH, _CH)],
                    sem,
                ))
            for c in copies:
                c.wait()
            pltpu.sync_copy(rows_v, out_hbm.at[pl.ds(base + g * _GRP, _GRP)])
            return carry

        lax.fori_loop(0, n_grp, grp_body, 0)

    return lookup


def kernel(token_ids, weight):
    b0, b1 = token_ids.shape
    num_emb, dim = weight.shape
    batch = b0 * b1
    flat_ids = token_ids.reshape(batch).astype(jnp.int32)
    out = _make_lookup(num_emb, dim, batch)(flat_ids, weight)
    return out.reshape(b0, b1, dim)
